# baseline (speedup 1.0000x reference)
import jax
import jax.numpy as jnp
from jax import lax
from jax.experimental import pallas as pl
from jax.experimental.pallas import tpu as pltpu
from jax.experimental.pallas import tpu_sc as plsc

_N = 8192
_SPLIT = 4096
_BR = 256

_NC = 2
_NS = 16
_NW = _NC * _NS
_RPW = (_N - _SPLIT) // _NW
_CR = 1
_NCHUNK = _RPW // _CR
_NBUF = 8
_PF = 4


def _tc_body(x_ref, v_ref, out_ref):
    g = pl.program_id(0)
    out_ref[...] = x_ref[...]
    vblock = v_ref[0, pl.ds(g * _BR, _BR)]
    rows = lax.broadcasted_iota(jnp.int32, (_BR, _BR), 0)
    cols = lax.broadcasted_iota(jnp.int32, (_BR, _BR), 1)
    diag = jnp.where(rows == cols, vblock[:, None], 0.0)
    out_ref[:, pl.ds(g * _BR, _BR)] = x_ref[:, pl.ds(g * _BR, _BR)] + diag


def _sc_body(x_hbm, v_hbm, out_hbm, buf, vals, in_sems, out_sems):
    wid = lax.axis_index("c") * _NS + lax.axis_index("s")
    r0 = _SPLIT + wid * _RPW
    o0 = wid * _RPW

    pltpu.make_async_copy(
        v_hbm.at[pl.ds(r0, _RPW)], vals.at[pl.ds(0, _RPW)], in_sems.at[0]
    ).start()
    pltpu.make_async_copy(
        v_hbm.at[pl.ds(r0, _RPW)], vals.at[pl.ds(0, _RPW)], in_sems.at[0]
    ).wait()

    def start_in(c, b):
        pltpu.make_async_copy(
            x_hbm.at[pl.ds(r0 + c * _CR, _CR), :], buf.at[b], in_sems.at[b]
        ).start()

    def wait_in(b):
        pltpu.make_async_copy(
            x_hbm.at[pl.ds(r0, _CR), :], buf.at[b], in_sems.at[b]
        ).wait()

    def start_out(c, b):
        pltpu.make_async_copy(
            buf.at[b], out_hbm.at[pl.ds(o0 + c * _CR, _CR), :], out_sems.at[b]
        ).start()

    def wait_out(b):
        pltpu.make_async_copy(
            buf.at[b], out_hbm.at[pl.ds(o0, _CR), :], out_sems.at[b]
        ).wait()

    for b in range(_PF):
        start_in(b, b)

    iota = lax.broadcasted_iota(jnp.int32, (16,), 0)
    diag_mask = iota < _CR

    def outer(o, _):
        for b in range(_NBUF):
            c = o * _NBUF + b
            wait_in(b)
            vals_v = plsc.load_gather(vals, [c * _CR + iota])
            col0 = r0 + c * _CR
            plsc.addupdate_scatter(
                buf.at[b], [iota, col0 + iota], vals_v, mask=diag_mask
            )
            start_out(c, b)
            nb = (b + _PF) % _NBUF

            @pl.when(c + _PF < _NCHUNK)
            def _():
                @pl.when(c + _PF >= _NBUF)
                def _():
                    wait_out(nb)

                start_in(c + _PF, nb)

        return ()

    lax.fori_loop(0, _NCHUNK // _NBUF, outer, ())

    for b in range(_NBUF):
        wait_out(b)


def kernel(x, values):
    v2d = values.reshape(1, _N)
    top = pl.pallas_call(
        _tc_body,
        grid=(_SPLIT // _BR,),
        in_specs=[
            pl.BlockSpec((_BR, _N), lambda g: (g, 0)),
            pl.BlockSpec((1, _N), lambda g: (0, 0)),
        ],
        out_specs=pl.BlockSpec((_BR, _N), lambda g: (g, 0)),
        out_shape=jax.ShapeDtypeStruct((_SPLIT, _N), x.dtype),
    )(x, v2d)

    mesh = plsc.VectorSubcoreMesh(
        core_axis_name="c", subcore_axis_name="s", num_cores=_NC, num_subcores=_NS
    )
    bottom = pl.kernel(
        _sc_body,
        out_type=jax.ShapeDtypeStruct((_N - _SPLIT, _N), jnp.float32),
        mesh=mesh,
        scratch_types=[
            pltpu.VMEM((_NBUF, _CR, _N), jnp.float32),
            pltpu.VMEM((_RPW + 16,), jnp.float32),
            pltpu.SemaphoreType.DMA((_NBUF,)),
            pltpu.SemaphoreType.DMA((_NBUF,)),
        ],
        compiler_params=pltpu.CompilerParams(needs_layout_passes=False),
    )(x, values)

    return jnp.concatenate([top, bottom], axis=0)


# speedup vs baseline: 1.8629x; 1.8629x over previous
import jax
import jax.numpy as jnp
from jax import lax
from jax.experimental import pallas as pl
from jax.experimental.pallas import tpu as pltpu
from jax.experimental.pallas import tpu_sc as plsc

_N = 8192
_NC = 2
_NS = 16
_NW = _NC * _NS
_RPW = _N // _NW
_CR = 4
_NCHUNK = _RPW // _CR
_NBUF = 2


def _win_start(r0, c):
    base = r0 + c * _CR
    beta = lax.rem(base, 128)
    return pl.multiple_of(base - beta, 128), beta


def _sc_body(x_hbm, v_hbm, out_hbm, buf, win, vals, in_sems, out_sems, wi_sems, wo_sems):
    sid = lax.axis_index("s")
    wid = lax.axis_index("c") * _NS + sid
    r0 = wid * _RPW

    pltpu.make_async_copy(
        v_hbm.at[pl.ds(r0, _RPW)], vals.at[pl.ds(0, _RPW)], in_sems.at[0]
    ).start()
    pltpu.make_async_copy(
        v_hbm.at[pl.ds(r0, _RPW)], vals.at[pl.ds(0, _RPW)], in_sems.at[0]
    ).wait()

    def start_in(c, b):
        pltpu.make_async_copy(
            x_hbm.at[pl.ds(r0 + c * _CR, _CR), :], buf.at[sid, b], in_sems.at[b]
        ).start()
        w, _ = _win_start(r0, c)
        pltpu.make_async_copy(
            x_hbm.at[pl.ds(r0 + c * _CR, _CR), pl.ds(w, 128)],
            win.at[b],
            wi_sems.at[b],
        ).start()

    def wait_in(b):
        pltpu.make_async_copy(
            x_hbm.at[pl.ds(r0, _CR), :], buf.at[sid, b], in_sems.at[b]
        ).wait()
        pltpu.make_async_copy(
            x_hbm.at[pl.ds(r0, _CR), pl.ds(0, 128)], win.at[b], wi_sems.at[b]
        ).wait()

    def start_out(c, b):
        pltpu.make_async_copy(
            buf.at[sid, b], out_hbm.at[pl.ds(r0 + c * _CR, _CR), :], out_sems.at[b]
        ).start()

    def wait_out(b):
        pltpu.make_async_copy(
            buf.at[sid, b], out_hbm.at[pl.ds(r0, _CR), :], out_sems.at[b]
        ).wait()

    start_in(0, 0)

    iota = lax.broadcasted_iota(jnp.int32, (16,), 0)
    diag_mask = iota < _CR

    def outer(o, _):
        for b in range(_NBUF):
            c = o * _NBUF + b
            wait_in(b)
            w, beta = _win_start(r0, c)
            vals_v = plsc.load_gather(vals, [c * _CR + iota])
            plsc.addupdate_scatter(
                win.at[b], [iota, beta + iota], vals_v, mask=diag_mask
            )
            pltpu.make_async_copy(
                win.at[b], buf.at[sid, b, :, pl.ds(w, 128)], wo_sems.at[b]
            ).start()
            pltpu.make_async_copy(
                win.at[b], buf.at[sid, b, :, pl.ds(0, 128)], wo_sems.at[b]
            ).wait()
            start_out(c, b)
            nb = (b + 1) % _NBUF

            @pl.when(c + 1 < _NCHUNK)
            def _():
                @pl.when(c >= 1)
                def _():
                    wait_out(nb)

                start_in(c + 1, nb)

        return ()

    lax.fori_loop(0, _NCHUNK // _NBUF, outer, ())

    for b in range(_NBUF):
        wait_out(b)


def kernel(x, values):
    mesh = plsc.VectorSubcoreMesh(
        core_axis_name="c", subcore_axis_name="s", num_cores=_NC, num_subcores=_NS
    )
    f = pl.kernel(
        _sc_body,
        out_type=jax.ShapeDtypeStruct((_N, _N), jnp.float32),
        mesh=mesh,
        scratch_types=[
            pltpu.MemorySpace.VMEM_SHARED((_NS, _NBUF, _CR, _N), jnp.float32),
            pltpu.VMEM((_NBUF, _CR, 128), jnp.float32),
            pltpu.VMEM((_RPW + 16,), jnp.float32),
            pltpu.SemaphoreType.DMA((_NBUF,)),
            pltpu.SemaphoreType.DMA((_NBUF,)),
            pltpu.SemaphoreType.DMA((_NBUF,)),
            pltpu.SemaphoreType.DMA((_NBUF,)),
        ],
        compiler_params=pltpu.CompilerParams(needs_layout_passes=False),
    )
    return f(x, values)


# speedup vs baseline: 1.8735x; 1.0057x over previous
import jax
import jax.numpy as jnp
from jax import lax
from jax.experimental import pallas as pl
from jax.experimental.pallas import tpu as pltpu
from jax.experimental.pallas import tpu_sc as plsc

_N = 8192
_NC = 2
_NS = 16
_NW = _NC * _NS
_RPW = _N // _NW
_CR = 2
_NCHUNK = _RPW // _CR
_NBUF = 4
_PF = 2


def _win_start(r0, c):
    base = r0 + c * _CR
    beta = lax.rem(base, 128)
    return pl.multiple_of(base - beta, 128), beta


def _sc_body(x_hbm, v_hbm, out_hbm, buf, win, vals, in_sems, out_sems, wi_sems, wo_sems):
    sid = lax.axis_index("s")
    wid = lax.axis_index("c") * _NS + sid
    r0 = wid * _RPW

    pltpu.make_async_copy(
        v_hbm.at[pl.ds(r0, _RPW)], vals.at[pl.ds(0, _RPW)], in_sems.at[0]
    ).start()
    pltpu.make_async_copy(
        v_hbm.at[pl.ds(r0, _RPW)], vals.at[pl.ds(0, _RPW)], in_sems.at[0]
    ).wait()

    def start_in(c, b):
        pltpu.make_async_copy(
            x_hbm.at[pl.ds(r0 + c * _CR, _CR), :], buf.at[sid, b], in_sems.at[b]
        ).start()
        w, _ = _win_start(r0, c)
        pltpu.make_async_copy(
            x_hbm.at[pl.ds(r0 + c * _CR, _CR), pl.ds(w, 128)],
            win.at[b],
            wi_sems.at[b],
        ).start()

    def wait_in(b):
        pltpu.make_async_copy(
            x_hbm.at[pl.ds(r0, _CR), :], buf.at[sid, b], in_sems.at[b]
        ).wait()
        pltpu.make_async_copy(
            x_hbm.at[pl.ds(r0, _CR), pl.ds(0, 128)], win.at[b], wi_sems.at[b]
        ).wait()

    def start_out(c, b):
        pltpu.make_async_copy(
            buf.at[sid, b], out_hbm.at[pl.ds(r0 + c * _CR, _CR), :], out_sems.at[b]
        ).start()

    def wait_out(b):
        pltpu.make_async_copy(
            buf.at[sid, b], out_hbm.at[pl.ds(r0, _CR), :], out_sems.at[b]
        ).wait()

    for b in range(_PF):
        start_in(b, b)

    iota = lax.broadcasted_iota(jnp.int32, (16,), 0)
    diag_mask = iota < _CR

    def outer(o, _):
        for b in range(_NBUF):
            c = o * _NBUF + b
            wait_in(b)
            w, beta = _win_start(r0, c)
            vals_v = plsc.load_gather(vals, [c * _CR + iota])
            plsc.addupdate_scatter(
                win.at[b], [iota, beta + iota], vals_v, mask=diag_mask
            )
            pltpu.make_async_copy(
                win.at[b], buf.at[sid, b, :, pl.ds(w, 128)], wo_sems.at[b]
            ).start()
            pltpu.make_async_copy(
                win.at[b], buf.at[sid, b, :, pl.ds(0, 128)], wo_sems.at[b]
            ).wait()
            start_out(c, b)
            nb = (b + _PF) % _NBUF

            @pl.when(c + _PF < _NCHUNK)
            def _():
                @pl.when(c + _PF >= _NBUF)
                def _():
                    wait_out(nb)

                start_in(c + _PF, nb)

        return ()

    lax.fori_loop(0, _NCHUNK // _NBUF, outer, ())

    for b in range(_NBUF):
        wait_out(b)


def kernel(x, values):
    mesh = plsc.VectorSubcoreMesh(
        core_axis_name="c", subcore_axis_name="s", num_cores=_NC, num_subcores=_NS
    )
    f = pl.kernel(
        _sc_body,
        out_type=jax.ShapeDtypeStruct((_N, _N), jnp.float32),
        mesh=mesh,
        scratch_types=[
            pltpu.MemorySpace.VMEM_SHARED((_NS, _NBUF, _CR, _N), jnp.float32),
            pltpu.VMEM((_NBUF, _CR, 128), jnp.float32),
            pltpu.VMEM((_RPW + 16,), jnp.float32),
            pltpu.SemaphoreType.DMA((_NBUF,)),
            pltpu.SemaphoreType.DMA((_NBUF,)),
            pltpu.SemaphoreType.DMA((_NBUF,)),
            pltpu.SemaphoreType.DMA((_NBUF,)),
        ],
        compiler_params=pltpu.CompilerParams(needs_layout_passes=False),
    )
    return f(x, values)
